# baseline (speedup 1.0000x reference)
import jax
import jax.numpy as jnp
from jax.experimental import pallas as pl

MAX_TRACK = 256
SCORE_THRESH = 0.7
FILTER_SCORE_THRESH = 0.6
MISS_TOLERANCE = 5

_N = 64 * 80
_R, _C = 40, 128


def _mask_kernel(cls_ref, obj_ref, dis_ref, mq_ref, msc_ref, sc_ref):
    cls = cls_ref[...]
    scores = 1.0 / (1.0 + jnp.exp(-cls))
    obj = obj_ref[...]
    dis = dis_ref[...]
    mq = mq_ref[...]
    newly = (obj == -1) & (scores >= SCORE_THRESH)
    dropped = (scores < FILTER_SCORE_THRESH) & (dis + 1 >= MISS_TOLERANCE)
    active = (obj >= 0) & jnp.logical_not(dropped)
    valid = (mq == 1) & (newly | active)
    msc_ref[...] = jnp.where(valid, scores, -jnp.inf)
    sc_ref[...] = scores


def kernel(out_hs, outputs_classes_head, outputs_coords_head, obj_idxes,
           disappear_time, mask_query):
    cls = outputs_classes_head.reshape(_R, _C)
    obj = obj_idxes.reshape(_R, _C)
    dis = disappear_time.reshape(_R, _C)
    mq = mask_query.reshape(_R, _C)

    masked_scores, scores = pl.pallas_call(
        _mask_kernel,
        out_shape=[
            jax.ShapeDtypeStruct((_R, _C), jnp.float32),
            jax.ShapeDtypeStruct((_R, _C), jnp.float32),
        ],
    )(cls, obj, dis, mq)

    masked_scores = masked_scores.reshape(_N)
    topk_scores, topk_idx = jax.lax.top_k(masked_scores, MAX_TRACK)

    hs = out_hs.reshape(256, _N)
    coord = outputs_coords_head.reshape(4, _N)
    cls_flat = outputs_classes_head.reshape(_N)

    sel_emb = hs[:, topk_idx].T
    sel_boxes = jax.nn.sigmoid(coord[:, topk_idx].T)
    sel_logits = cls_flat[topk_idx][:, None]

    out = jnp.concatenate(
        [topk_scores[:, None], sel_boxes, sel_logits, sel_emb], axis=1)
    return out, topk_idx
